# speedup vs baseline: 3.2409x; 3.2409x over previous
import functools

import jax
import jax.numpy as jnp
from jax import lax
from jax.experimental import pallas as pl
from jax.experimental.pallas import tpu as pltpu
from jax.experimental.pallas import tpu_sc as plsc

NUM_EMB = 100000
D = 128
B_TOTAL = 4096 * 50
NC, NS = 2, 16
NW = NC * NS
BPW = B_TOTAL // NW
CHUNK = 128
NSTEP = BPW // CHUNK
NBUF = 2
NGRP = NSTEP // NBUF


def _emb_body(x_hbm, w_hbm, out_hbm, idx_v, rows_v, gsems, osems):
    wid = lax.axis_index("s") * NC + lax.axis_index("c")
    base = wid * BPW

    pltpu.sync_copy(x_hbm.at[wid], idx_v)

    def gstart(b, step):
        pltpu.async_copy(w_hbm.at[idx_v.at[step]], rows_v.at[b], gsems.at[b])

    def gwait(b):
        pltpu.make_async_copy(w_hbm.at[idx_v.at[0]], rows_v.at[b],
                              gsems.at[b]).wait()

    def wstart(b, step):
        pltpu.async_copy(rows_v.at[b],
                         out_hbm.at[pl.ds(base + step * CHUNK, CHUNK)],
                         osems.at[b])

    def wwait(b):
        pltpu.make_async_copy(rows_v.at[b],
                              out_hbm.at[pl.ds(base, CHUNK)],
                              osems.at[b]).wait()

    for b in range(NBUF):
        gstart(b, b)

    def group(g, _):
        for b in range(NBUF):
            gwait(b)
            wstart(b, g * NBUF + b)
        for b in range(NBUF):
            wwait(b)
            nxt = (g + 1) * NBUF + b

            @pl.when(g < NGRP - 1)
            def _():
                gstart(b, nxt)
        return _

    lax.fori_loop(0, NGRP, group, None)


@jax.jit
def _emb(xw, weight):
    kern = pl.kernel(
        _emb_body,
        out_type=jax.ShapeDtypeStruct((B_TOTAL, D), jnp.float32),
        mesh=plsc.VectorSubcoreMesh(core_axis_name="c", subcore_axis_name="s"),
        scratch_types=[
            pltpu.VMEM((NSTEP, CHUNK), jnp.int32),
            pltpu.VMEM((NBUF, CHUNK, D), jnp.float32),
            pltpu.SemaphoreType.DMA((NBUF,)),
            pltpu.SemaphoreType.DMA((NBUF,)),
        ],
    )
    return kern(xw, weight)


def kernel(x, weight):
    xw = x.astype(jnp.int32).reshape(NW, NSTEP, CHUNK)
    out = _emb(xw, weight)
    return out.reshape(4096, 50, D)


# speedup vs baseline: 3.3110x; 1.0216x over previous
import functools

import jax
import jax.numpy as jnp
from jax import lax
from jax.experimental import pallas as pl
from jax.experimental.pallas import tpu as pltpu
from jax.experimental.pallas import tpu_sc as plsc

NUM_EMB = 100000
D = 128
B_TOTAL = 4096 * 50
NC, NS = 2, 16
NW = NC * NS
BPW = B_TOTAL // NW
CHUNK = 128
NSTEP = BPW // CHUNK
NBUF = 5
NGRP = NSTEP // NBUF


def _emb_body(x_hbm, w_hbm, out_hbm, idx_v, rows_v, gsems, osems):
    wid = lax.axis_index("s") * NC + lax.axis_index("c")
    base = wid * BPW

    pltpu.sync_copy(x_hbm.at[wid], idx_v)

    def gstart(b, step):
        pltpu.async_copy(w_hbm.at[idx_v.at[step]], rows_v.at[b], gsems.at[b])

    def gwait(b):
        pltpu.make_async_copy(w_hbm.at[idx_v.at[0]], rows_v.at[b],
                              gsems.at[b]).wait()

    def wstart(b, step):
        pltpu.async_copy(rows_v.at[b],
                         out_hbm.at[pl.ds(base + step * CHUNK, CHUNK)],
                         osems.at[b])

    def wwait(b):
        pltpu.make_async_copy(rows_v.at[b],
                              out_hbm.at[pl.ds(base, CHUNK)],
                              osems.at[b]).wait()

    for b in range(NBUF):
        gstart(b, b)

    def group(g, _):
        for b in range(NBUF):
            gwait(b)
            wstart(b, g * NBUF + b)
        for b in range(NBUF):
            wwait(b)
            nxt = (g + 1) * NBUF + b

            @pl.when(g < NGRP - 1)
            def _():
                gstart(b, nxt)
        return _

    lax.fori_loop(0, NGRP, group, None)


@jax.jit
def _emb(xw, weight):
    kern = pl.kernel(
        _emb_body,
        out_type=jax.ShapeDtypeStruct((B_TOTAL, D), jnp.float32),
        mesh=plsc.VectorSubcoreMesh(core_axis_name="c", subcore_axis_name="s"),
        scratch_types=[
            pltpu.VMEM((NSTEP, CHUNK), jnp.int32),
            pltpu.VMEM((NBUF, CHUNK, D), jnp.float32),
            pltpu.SemaphoreType.DMA((NBUF,)),
            pltpu.SemaphoreType.DMA((NBUF,)),
        ],
    )
    return kern(xw, weight)


def kernel(x, weight):
    xw = x.astype(jnp.int32).reshape(NW, NSTEP, CHUNK)
    out = _emb(xw, weight)
    return out.reshape(4096, 50, D)


# speedup vs baseline: 5.9472x; 1.7962x over previous
import functools

import jax
import jax.numpy as jnp
from jax import lax
from jax.experimental import pallas as pl
from jax.experimental.pallas import tpu as pltpu
from jax.experimental.pallas import tpu_sc as plsc

B, S, D = 4096, 50, 128
NC, NS = 2, 16
NW = NC * NS
BPW = B // NW
NSTEP = BPW
NBUF = 8
NGRP = NSTEP // NBUF


def _emb_body(x_hbm, w_hbm, out_hbm, idx_v, rows_v, gsems, osems):
    wid = lax.axis_index("s") * NC + lax.axis_index("c")
    b0 = wid * BPW

    pltpu.sync_copy(x_hbm.at[wid], idx_v)

    def gstart(b, step):
        pltpu.async_copy(w_hbm.at[idx_v.at[step]], rows_v.at[b], gsems.at[b])

    def gwait(b):
        pltpu.make_async_copy(w_hbm.at[idx_v.at[0]], rows_v.at[b],
                              gsems.at[b]).wait()

    def wstart(b, step):
        pltpu.async_copy(rows_v.at[b], out_hbm.at[b0 + step], osems.at[b])

    def wwait(b):
        pltpu.make_async_copy(rows_v.at[b], out_hbm.at[b0],
                              osems.at[b]).wait()

    for b in range(NBUF):
        gstart(b, b)

    def group(g, _):
        for b in range(NBUF):
            gwait(b)
            wstart(b, g * NBUF + b)
        for b in range(NBUF):
            wwait(b)
            nxt = (g + 1) * NBUF + b

            @pl.when(g < NGRP - 1)
            def _():
                gstart(b, nxt)
        return _

    lax.fori_loop(0, NGRP, group, None)


@jax.jit
def _emb(xw, weight):
    kern = pl.kernel(
        _emb_body,
        out_type=jax.ShapeDtypeStruct((B, S, D), jnp.float32),
        mesh=plsc.VectorSubcoreMesh(core_axis_name="c", subcore_axis_name="s"),
        scratch_types=[
            pltpu.VMEM((NSTEP, S), jnp.int32),
            pltpu.VMEM((NBUF, S, D), jnp.float32),
            pltpu.SemaphoreType.DMA((NBUF,)),
            pltpu.SemaphoreType.DMA((NBUF,)),
        ],
    )
    return kern(xw, weight)


def kernel(x, weight):
    xw = x.astype(jnp.int32).reshape(NW, NSTEP, S)
    return _emb(xw, weight)
